# baseline (speedup 1.0000x reference)
import functools

import jax
import jax.numpy as jnp
from jax import lax
from jax.experimental import pallas as pl
from jax.experimental.pallas import tpu as pltpu
from jax.experimental.pallas import tpu_sc as plsc

VOCAB = 1_000_000
EMBED = 16
BATCH = 16384

NC = 2
NS = 16
L = 16
NW = NC * NS
BPW = BATCH // NW
RD = 8
NBODY = BPW // RD
W = 128
PAD = 32


def _sc_body(x_hbm, t_hbm, tgt_hbm, ctx_hbm, out_hbm,
             xi_v, ti_v, ring_t, ring_x, rows_p, out_v, *sems):
    sem_t = sems[:RD]
    sem_x = sems[RD:]
    wid = lax.axis_index("s") * NC + lax.axis_index("c")
    base = wid * BPW

    pltpu.sync_copy(x_hbm.at[pl.ds(base, BPW)], xi_v.at[pl.ds(0, BPW)])
    pltpu.sync_copy(t_hbm.at[pl.ds(base, BPW)], ti_v.at[pl.ds(0, BPW)])

    lane = lax.iota(jnp.int32, L)
    gvec = lane >> 3
    svec = lane & 7

    def fire(k, vbt_k, vbx_k):
        pltpu.async_copy(
            tgt_hbm.at[:, :, pl.ds(pl.multiple_of(vbt_k, W), W)],
            ring_t.at[:, :, pl.ds(k * W, W)], sem_t[k])
        pltpu.async_copy(
            ctx_hbm.at[:, :, pl.ds(pl.multiple_of(vbx_k, W), W)],
            ring_x.at[:, :, pl.ds(k * W, W)], sem_x[k])

    vbt0 = (ti_v[pl.ds(0, L)] >> 7) * W
    vbx0 = (xi_v[pl.ds(0, L)] >> 7) * W
    for k in range(RD):
        fire(k, vbt0[k], vbx0[k])

    def body(j, carry):
        i0 = j * RD
        lt = ti_v[pl.ds(i0, L)] & (W - 1)
        lx = xi_v[pl.ds(i0, L)] & (W - 1)
        vbt = (ti_v[pl.ds(i0 + RD, L)] >> 7) * W
        vbx = (xi_v[pl.ds(i0 + RD, L)] >> 7) * W
        for k in range(RD):
            pltpu.make_async_copy(
                tgt_hbm.at[:, :, pl.ds(0, W)],
                ring_t.at[:, :, pl.ds(k * W, W)], sem_t[k]).wait()
            pltpu.make_async_copy(
                ctx_hbm.at[:, :, pl.ds(0, W)],
                ring_x.at[:, :, pl.ds(k * W, W)], sem_x[k]).wait()
            col_t = jnp.full((L,), k * W, jnp.int32) + lt[k]
            col_x = jnp.full((L,), k * W, jnp.int32) + lx[k]
            ert = plsc.load_gather(ring_t, [gvec, svec, col_t])
            erx = plsc.load_gather(ring_x, [gvec, svec, col_x])
            rows_p[i0 + k, :] = ert * erx

            @pl.when(j < NBODY - 1)
            def _():
                fire(k, vbt[k], vbx[k])
        return carry

    lax.fori_loop(0, NBODY, body, 0)

    def reduce(b, carry):
        i0 = b * L
        ridx = i0 + lane
        acc = jnp.zeros((L,), jnp.float32)
        for d in range(EMBED):
            acc = acc + plsc.load_gather(
                rows_p, [ridx, jnp.full((L,), d, jnp.int32)])
        out_v[pl.ds(i0, L)] = acc
        return carry

    lax.fori_loop(0, BPW // L, reduce, 0)

    pltpu.sync_copy(out_v, out_hbm.at[pl.ds(base, BPW)])


@jax.jit
def _skipgram_sc(x1, t1, tgt3, ctx3):
    mesh = plsc.VectorSubcoreMesh(core_axis_name="c", subcore_axis_name="s")
    k = functools.partial(
        pl.kernel,
        mesh=mesh,
        out_type=jax.ShapeDtypeStruct((BATCH,), jnp.float32),
        compiler_params=pltpu.CompilerParams(
            needs_layout_passes=False,
            use_tc_tiling_on_sc=True),
        scratch_types=[
            pltpu.VMEM((BPW + PAD,), jnp.int32),
            pltpu.VMEM((BPW + PAD,), jnp.int32),
            pltpu.VMEM((2, 8, RD * W), jnp.float32),
            pltpu.VMEM((2, 8, RD * W), jnp.float32),
            pltpu.VMEM((BPW, EMBED), jnp.float32),
            pltpu.VMEM((BPW,), jnp.float32),
        ] + [pltpu.SemaphoreType.DMA] * (2 * RD),
    )(_sc_body)
    return k(x1, t1, tgt3, ctx3)


def kernel(x, t, target_table, context_table):
    x1 = x.astype(jnp.int32)
    t1 = t.astype(jnp.int32)
    tgt3 = target_table.T.reshape(2, 8, VOCAB)
    ctx3 = context_table.T.reshape(2, 8, VOCAB)
    return _skipgram_sc(x1, t1, tgt3, ctx3)


# speedup vs baseline: 1.1250x; 1.1250x over previous
import functools

import jax
import jax.numpy as jnp
from jax import lax
from jax.experimental import pallas as pl
from jax.experimental.pallas import tpu as pltpu
from jax.experimental.pallas import tpu_sc as plsc

VOCAB = 1_000_000
EMBED = 16
BATCH = 16384

NC = 2
NS = 16
L = 16
NW = NC * NS
BPW = BATCH // NW
NGRP = BPW // L
W = 128


def _sc_body(x_hbm, t_hbm, tgt_hbm, ctx_hbm, out_hbm,
             xi_v, ti_v, tgt_stage, ctx_stage, out_v, sem_t, sem_c):
    wid = lax.axis_index("s") * NC + lax.axis_index("c")
    base = wid * BPW

    pltpu.sync_copy(x_hbm.at[pl.ds(wid * 2, 2)], xi_v)
    pltpu.sync_copy(t_hbm.at[pl.ds(wid * 2, 2)], ti_v)

    lane = lax.iota(jnp.int32, L)

    def chunk_body(c, carry):
        r = c // L
        o = (c % L) * L
        vx = xi_v[r, pl.ds(o, L)]
        vt = ti_v[r, pl.ds(o, L)]
        xcol = lane * W + (vx & (W - 1))
        tcol = lane * W + (vt & (W - 1))
        xb = (vx >> 7) * W
        tb = (vt >> 7) * W
        copies = []
        for k in range(L):
            copies.append(pltpu.async_copy(
                tgt_hbm.at[:, :, pl.ds(pl.multiple_of(tb[k], W), W)],
                tgt_stage.at[:, :, pl.ds(k * W, W)], sem_t))
            copies.append(pltpu.async_copy(
                ctx_hbm.at[:, :, pl.ds(pl.multiple_of(xb[k], W), W)],
                ctx_stage.at[:, :, pl.ds(k * W, W)], sem_c))
        for cp in copies:
            cp.wait()

        acc = jnp.zeros((L,), jnp.float32)
        for d in range(EMBED):
            gv = jnp.full((L,), d // 8, jnp.int32)
            sv = jnp.full((L,), d % 8, jnp.int32)
            tv = plsc.load_gather(tgt_stage, [gv, sv, tcol])
            cv = plsc.load_gather(ctx_stage, [gv, sv, xcol])
            acc = acc + tv * cv
        out_v[pl.ds(c * L, L)] = acc
        return carry

    lax.fori_loop(0, NGRP, chunk_body, 0)

    pltpu.sync_copy(out_v, out_hbm.at[pl.ds(base, BPW)])


@jax.jit
def _skipgram_sc(x2d, t2d, tgt3, ctx3):
    mesh = plsc.VectorSubcoreMesh(core_axis_name="c", subcore_axis_name="s")
    k = functools.partial(
        pl.kernel,
        mesh=mesh,
        out_type=jax.ShapeDtypeStruct((BATCH,), jnp.float32),
        compiler_params=pltpu.CompilerParams(
            needs_layout_passes=False,
            use_tc_tiling_on_sc=True),
        scratch_types=[
            pltpu.VMEM((2, 2 * W), jnp.int32),
            pltpu.VMEM((2, 2 * W), jnp.int32),
            pltpu.VMEM((2, 8, L * W), jnp.float32),
            pltpu.VMEM((2, 8, L * W), jnp.float32),
            pltpu.VMEM((BPW,), jnp.float32),
            pltpu.SemaphoreType.DMA,
            pltpu.SemaphoreType.DMA,
        ],
    )(_sc_body)
    return k(x2d, t2d, tgt3, ctx3)


def kernel(x, t, target_table, context_table):
    x2d = x.astype(jnp.int32).reshape(BATCH // (2 * W), 2 * W)
    t2d = t.astype(jnp.int32).reshape(BATCH // (2 * W), 2 * W)
    tgt3 = target_table.T.reshape(2, 8, VOCAB)
    ctx3 = context_table.T.reshape(2, 8, VOCAB)
    return _skipgram_sc(x2d, t2d, tgt3, ctx3)
